# baseline (speedup 1.0000x reference)
import functools

import jax
import jax.numpy as jnp
from jax import lax
from jax.experimental import pallas as pl
from jax.experimental.pallas import tpu as pltpu
from jax.experimental.pallas import tpu_sc as plsc

NC = 2
NS = 16
NW = NC * NS
L = 16
PW = 128
NBUF = 8
HALF = 256


def _make_sc_gather(V, D, B):
    bpw = B // NW

    mesh = plsc.VectorSubcoreMesh(core_axis_name="c", subcore_axis_name="s")

    @functools.partial(
        pl.kernel,
        mesh=mesh,
        compiler_params=pltpu.CompilerParams(needs_layout_passes=False),
        out_type=jax.ShapeDtypeStruct((D, B), jnp.float32),
        scratch_types=[
            pltpu.VMEM((bpw // PW, PW), jnp.int32),
            pltpu.VMEM((NBUF, D, PW), jnp.float32),
            pltpu.VMEM((D, HALF), jnp.float32),
            pltpu.VMEM((D, HALF), jnp.float32),
            [pltpu.SemaphoreType.DMA] * NBUF,
        ],
    )
    def gather_k(y_hbm, embT_hbm, zT_hbm, g_hbm, idx_v, panels_v, rows_v,
                 z_v, sems):
        wid = lax.axis_index("s") * NC + lax.axis_index("c")
        base = wid * bpw
        pltpu.sync_copy(y_hbm.at[wid], idx_v)

        lanes = lax.iota(jnp.int32, L)
        perms = [lax.bitwise_xor(lanes, jnp.int32(k)) for k in (8, 4, 2, 1)]

        def scalar_idx(r):
            g = lax.shift_right_logical(r, 4)
            vec = idx_v[lax.shift_right_logical(g, 3),
                        pl.ds(pl.multiple_of((g & 7) * L, L), L)]
            j = jnp.full((L,), r & (L - 1), jnp.int32)
            return vec.at[j].get(mode="promise_in_bounds")[0]

        def fire(r, buf, sem):
            i = scalar_idx(r)
            start = pl.multiple_of(i & ~jnp.int32(PW - 1), PW)
            pltpu.async_copy(
                embT_hbm.at[:, pl.ds(start, PW)], panels_v.at[buf], sem
            )

        def drain(buf, sem):
            pltpu.make_async_copy(
                embT_hbm.at[:, pl.ds(0, PW)], panels_v.at[buf], sem
            ).wait()

        def extract(r, buf):
            i = scalar_idx(r)
            col = jnp.full((L,), i & (PW - 1), jnp.int32)
            rl = jnp.full((L,), r & (HALF - 1), jnp.int32)
            zq = []
            for k in range(D // L):
                zq.append(plsc.load_gather(z_v, [lanes + (L * k), rl]))
            s_vec = zq[0] * zq[0]
            for k in range(1, D // L):
                s_vec = s_vec + zq[k] * zq[k]
            for perm in perms:
                s_vec = s_vec + s_vec.at[perm].get(mode="promise_in_bounds")
            iv = lax.bitcast_convert_type(s_vec, jnp.int32)
            iv = jnp.int32(0x5F3759DF) - lax.shift_right_logical(iv, 1)
            yv = lax.bitcast_convert_type(iv, jnp.float32)
            half_s = s_vec * 0.5
            for _ in range(3):
                yv = yv * (1.5 - half_s * yv * yv)
            for k in range(D // L):
                row_idx = lanes + (L * k)
                q = plsc.load_gather(panels_v.at[buf], [row_idx, col])
                plsc.store_scatter(rows_v, [row_idx, rl], q + zq[k] * yv)

        for h in range(bpw // HALF):
            r_lo = h * HALF
            r_hi = r_lo + HALF
            pltpu.sync_copy(
                zT_hbm.at[:, pl.ds(pl.multiple_of(base + r_lo, HALF), HALF)],
                z_v,
            )
            for p in range(NBUF - 1):
                fire(jnp.int32(r_lo + p), p, sems[p])

            def quad_body(rq, carry):
                r0 = r_lo + rq * NBUF
                for p in range(NBUF):
                    r = r0 + p
                    nb = (p + NBUF - 1) % NBUF

                    @pl.when(r + NBUF - 1 < r_hi)
                    def _():
                        fire(r + NBUF - 1, nb, sems[nb])

                    drain(p, sems[p])
                    extract(r, p)
                return carry

            lax.fori_loop(0, HALF // NBUF, quad_body, 0)
            pltpu.sync_copy(
                rows_v,
                g_hbm.at[:, pl.ds(pl.multiple_of(base + r_lo, HALF), HALF)],
            )

    return gather_k


def kernel(z, y, embedding):
    B, D = z.shape
    V = embedding.shape[0]
    bpw = B // NW
    y3 = y.astype(jnp.int32).reshape(NW, bpw // PW, PW)
    outT = _make_sc_gather(V, D, B)(y3, embedding.T, z.T)
    return outT.T
